# baseline (speedup 1.0000x reference)
import functools

import jax
import jax.numpy as jnp
from jax import lax
from jax.experimental import pallas as pl
from jax.experimental.pallas import tpu as pltpu
from jax.experimental.pallas import tpu_sc as plsc

N = 10000
E = 320000
D = 128
H = 128
C = 16
G = 16

NP = 10240
ROWS_PER_TILE = 640
CHUNK = 128
NCHUNKS = 2560
EP = NCHUNKS * CHUNK
Q_FAST = 160
CPT_DEG = 80
RB = 1024
GRID = NP // RB

_mesh = plsc.VectorSubcoreMesh(core_axis_name="c", subcore_axis_name="s")


@functools.partial(
    pl.kernel,
    mesh=_mesh,
    out_type=jax.ShapeDtypeStruct((2, NP), jnp.float32),
    scratch_types=[
        pltpu.VMEM((CPT_DEG, CHUNK), jnp.int32),
        pltpu.VMEM((CHUNK,), jnp.float32),
        pltpu.VMEM((ROWS_PER_TILE,), jnp.float32),
        pltpu.VMEM_SHARED((NP,), jnp.float32),
        pltpu.SemaphoreType.DMA,
    ],
)
def _sc_degree(dst_hbm, ones_hbm, z640_hbm, out_hbm,
               dst_v, ones_v, stage_v, deg_sh, sem):
    c = lax.axis_index("c")
    s = lax.axis_index("s")
    wid = c * 16 + s
    row0 = s * ROWS_PER_TILE
    pltpu.sync_copy(dst_hbm.at[wid], dst_v)
    pltpu.sync_copy(ones_hbm, ones_v)
    pltpu.sync_copy(z640_hbm, stage_v)
    pltpu.sync_copy(stage_v, deg_sh.at[pl.ds(row0, ROWS_PER_TILE)])
    plsc.subcore_barrier()

    def body(j, carry):
        pltpu.sync_copy(ones_v, deg_sh.at[dst_v.at[j]], add=True)
        return carry

    lax.fori_loop(0, CPT_DEG, body, 0)
    plsc.subcore_barrier()
    pltpu.sync_copy(deg_sh.at[pl.ds(row0, ROWS_PER_TILE)], stage_v)
    pltpu.sync_copy(stage_v, out_hbm.at[c, pl.ds(row0, ROWS_PER_TILE)])


@functools.partial(
    pl.kernel,
    mesh=_mesh,
    out_type=jax.ShapeDtypeStruct((NP, H), jnp.float32),
    scratch_types=[
        pltpu.VMEM((2, CHUNK), jnp.int32),
        pltpu.VMEM((2, CHUNK), jnp.int32),
        pltpu.VMEM((CHUNK, H), jnp.float32),
        pltpu.VMEM((CHUNK, H), jnp.float32),
        pltpu.SemaphoreType.DMA,
        pltpu.SemaphoreType.DMA,
        pltpu.SemaphoreType.DMA,
        pltpu.SemaphoreType.DMA,
        pltpu.SemaphoreType.DMA,
        pltpu.SemaphoreType.DMA,
        pltpu.VMEM_SHARED((NP, H), jnp.float32),
    ],
)
def _sc_aggregate(hs_hbm, src_hbm, dst_hbm, z_hbm, out_hbm,
                  src2, dst2, rows0, rows1, sg0, sg1, si0, si1, sd0, sd1,
                  acc_sh):
    c = lax.axis_index("c")
    s = lax.axis_index("s")
    row0 = s * ROWS_PER_TILE
    NZ = ROWS_PER_TILE // CHUNK

    run = c == 0
    qh = Q_FAST // 2
    base = s * Q_FAST

    @pl.when(run)
    def _():
        pltpu.sync_copy(z_hbm, rows0)
        for k in range(NZ):
            pltpu.sync_copy(rows0, acc_sh.at[pl.ds(row0 + k * CHUNK, CHUNK), :])
    plsc.subcore_barrier()

    def body(g, carry):
        j0 = base + 2 * g
        last = g == qh - 1
        pltpu.make_async_copy(src_hbm.at[base + 1], src2.at[1], si1).wait()
        pltpu.async_copy(hs_hbm.at[src2.at[1]], rows1, sg1)
        pltpu.make_async_copy(hs_hbm.at[src2.at[0]], rows0, sg0).wait()

        @pl.when(jnp.logical_not(last))
        def _():
            pltpu.async_copy(src_hbm.at[j0 + 2], src2.at[0], si0)

        pltpu.make_async_copy(dst_hbm.at[base], dst2.at[0], sd0).wait()
        pltpu.sync_copy(rows0, acc_sh.at[dst2.at[0]], add=True)

        @pl.when(jnp.logical_not(last))
        def _():
            pltpu.async_copy(dst_hbm.at[j0 + 2], dst2.at[0], sd0)
            pltpu.make_async_copy(src_hbm.at[j0 + 2], src2.at[0], si0).wait()
            pltpu.async_copy(hs_hbm.at[src2.at[0]], rows0, sg0)

        pltpu.make_async_copy(hs_hbm.at[src2.at[1]], rows1, sg1).wait()

        @pl.when(jnp.logical_not(last))
        def _():
            pltpu.async_copy(src_hbm.at[j0 + 3], src2.at[1], si1)

        pltpu.make_async_copy(dst_hbm.at[base + 1], dst2.at[1], sd1).wait()
        pltpu.sync_copy(rows1, acc_sh.at[dst2.at[1]], add=True)

        @pl.when(jnp.logical_not(last))
        def _():
            pltpu.async_copy(dst_hbm.at[j0 + 3], dst2.at[1], sd1)

        return carry

    @pl.when(run)
    def _():
        pltpu.sync_copy(src_hbm.at[base], src2.at[0])
        pltpu.async_copy(hs_hbm.at[src2.at[0]], rows0, sg0)
        pltpu.async_copy(dst_hbm.at[base], dst2.at[0], sd0)
        pltpu.async_copy(src_hbm.at[base + 1], src2.at[1], si1)
        pltpu.async_copy(dst_hbm.at[base + 1], dst2.at[1], sd1)
        lax.fori_loop(0, qh, body, 0)

    plsc.subcore_barrier()

    @pl.when(run)
    def _():
        bufs = (rows0, rows1)
        sems = (sg0, sg1)
        for k in range(NZ):
            sl = pl.ds(row0 + k * CHUNK, CHUNK)
            if k >= 2:
                prev = pl.ds(row0 + (k - 2) * CHUNK, CHUNK)
                pltpu.make_async_copy(
                    bufs[k % 2], out_hbm.at[prev, :], sems[k % 2]).wait()
            pltpu.sync_copy(acc_sh.at[sl, :], bufs[k % 2])
            pltpu.async_copy(bufs[k % 2], out_hbm.at[sl, :], sems[k % 2])
        for k in (NZ - 2, NZ - 1):
            sl = pl.ds(row0 + k * CHUNK, CHUNK)
            pltpu.make_async_copy(bufs[k % 2], out_hbm.at[sl, :], sems[k % 2]).wait()


def _t1_body(x_ref, w_ref, deg_ref, hs_ref):
    dinv = lax.rsqrt(deg_ref[0, :] + deg_ref[1, :] + 1.0)
    h = jnp.dot(x_ref[...], w_ref[...], preferred_element_type=jnp.float32)
    hs_ref[...] = h * dinv[:, None]


_t1 = pl.pallas_call(
    _t1_body,
    grid=(GRID,),
    in_specs=[
        pl.BlockSpec((RB, D), lambda i: (i, 0)),
        pl.BlockSpec((D, H), lambda i: (0, 0)),
        pl.BlockSpec((2, RB), lambda i: (0, i)),
    ],
    out_specs=pl.BlockSpec((RB, H), lambda i: (i, 0)),
    out_shape=jax.ShapeDtypeStruct((NP, H), jnp.float32),
)


def _t2_body(acc_ref, hs_ref, deg_ref, w_ref, b_ref, out_ref):
    dinv = lax.rsqrt(deg_ref[0, :] + deg_ref[1, :] + 1.0)
    tot = (acc_ref[...] + hs_ref[...]) * dinv[:, None]
    h = jnp.maximum(tot + b_ref[...][None, :], 0.0)
    out_ref[...] = (
        jnp.dot(h, w_ref[...], preferred_element_type=jnp.float32) * dinv[:, None]
    )


_t2 = pl.pallas_call(
    _t2_body,
    grid=(GRID,),
    in_specs=[
        pl.BlockSpec((RB, H), lambda i: (i, 0)),
        pl.BlockSpec((RB, H), lambda i: (i, 0)),
        pl.BlockSpec((2, RB), lambda i: (0, i)),
        pl.BlockSpec((H, H), lambda i: (0, 0)),
        pl.BlockSpec((H,), lambda i: (0,)),
    ],
    out_specs=pl.BlockSpec((RB, H), lambda i: (i, 0)),
    out_shape=jax.ShapeDtypeStruct((NP, H), jnp.float32),
)


def _t3_body(acc_ref, hs_ref, deg_ref, b_ref, batch_ref, wc_ref, bc_ref,
             out_ref, hg_acc, cnt_acc):
    i = pl.program_id(0)

    @pl.when(i == 0)
    def _():
        hg_acc[...] = jnp.zeros_like(hg_acc)
        cnt_acc[...] = jnp.zeros_like(cnt_acc)

    dinv = lax.rsqrt(deg_ref[0, :] + deg_ref[1, :] + 1.0)
    tot = (acc_ref[...] + hs_ref[...]) * dinv[:, None]
    h = jnp.maximum(tot + b_ref[...][None, :], 0.0)
    onehot = (
        batch_ref[0, 0, :][None, :]
        == lax.broadcasted_iota(jnp.int32, (G, RB), 0)
    ).astype(jnp.float32)
    hg_acc[...] += jnp.dot(onehot, h, preferred_element_type=jnp.float32)
    cnt = jnp.sum(onehot, axis=1, keepdims=True)
    cnt_acc[...] += jnp.broadcast_to(cnt, (G, H))

    @pl.when(i == GRID - 1)
    def _():
        hg = hg_acc[...] / jnp.maximum(cnt_acc[...], 1.0)
        out_ref[...] = (
            jnp.dot(hg, wc_ref[...], preferred_element_type=jnp.float32)
            + bc_ref[...][None, :]
        )


_t3 = pl.pallas_call(
    _t3_body,
    grid=(GRID,),
    in_specs=[
        pl.BlockSpec((RB, H), lambda i: (i, 0)),
        pl.BlockSpec((RB, H), lambda i: (i, 0)),
        pl.BlockSpec((2, RB), lambda i: (0, i)),
        pl.BlockSpec((H,), lambda i: (0,)),
        pl.BlockSpec((1, 1, RB), lambda i: (i, 0, 0)),
        pl.BlockSpec((H, H), lambda i: (0, 0)),
        pl.BlockSpec((H,), lambda i: (0,)),
    ],
    out_specs=pl.BlockSpec((G, H), lambda i: (0, 0)),
    out_shape=jax.ShapeDtypeStruct((G, H), jnp.float32),
    scratch_shapes=[
        pltpu.VMEM((G, H), jnp.float32),
        pltpu.VMEM((G, H), jnp.float32),
    ],
)


def kernel(x, edge_index, batch, W1, b1, W2, b2, Wc, bc):
    src = edge_index[0]
    dst = edge_index[1]
    pad_e = EP - E
    pad_dst = N + (jnp.arange(pad_e, dtype=jnp.int32) % (NP - N))
    src_p = jnp.concatenate(
        [src, jnp.zeros((pad_e,), jnp.int32)]).reshape(NCHUNKS, CHUNK)
    dst_p = jnp.concatenate([dst, pad_dst]).reshape(NCHUNKS, CHUNK)
    dst_p32 = dst_p.reshape(32, CPT_DEG, CHUNK)
    x_p = jnp.pad(x, ((0, NP - N), (0, 0)))
    batch_p = jnp.pad(batch, (0, NP - N), constant_values=G).reshape(GRID, 1, RB)
    wc_p = jnp.pad(Wc, ((0, 0), (0, H - C)))
    bc_p = jnp.pad(bc, (0, H - C))
    ones128 = jnp.ones((CHUNK,), jnp.float32)
    z640 = jnp.zeros((ROWS_PER_TILE,), jnp.float32)
    z128 = jnp.zeros((CHUNK, H), jnp.float32)

    deg2 = _sc_degree(dst_p32, ones128, z640)
    hs1 = _t1(x_p, W1, deg2)
    acc1 = _sc_aggregate(hs1, src_p, dst_p, z128)
    hs2 = _t2(acc1, hs1, deg2, W2, b1)
    acc2 = _sc_aggregate(hs2, src_p, dst_p, z128)
    out = _t3(acc2, hs2, deg2, b2, batch_p, wc_p, bc_p)
    return out[:, :C]


# speedup vs baseline: 3.7447x; 3.7447x over previous
import functools

import jax
import jax.numpy as jnp
from jax import lax
from jax.experimental import pallas as pl
from jax.experimental.pallas import tpu as pltpu
from jax.experimental.pallas import tpu_sc as plsc

N = 10000
E = 320000
D = 128
H = 128
C = 16
G = 16

NP = 10240
ROWS_PER_TILE = 640
CHUNK = 128
NCHUNKS = 2560
EP = NCHUNKS * CHUNK
QPT = 80
CPT_DEG = 80
RB = 1024
GRID = NP // RB

_mesh = plsc.VectorSubcoreMesh(core_axis_name="c", subcore_axis_name="s")


@functools.partial(
    pl.kernel,
    mesh=_mesh,
    out_type=jax.ShapeDtypeStruct((2, NP), jnp.float32),
    scratch_types=[
        pltpu.VMEM((CPT_DEG, CHUNK), jnp.int32),
        pltpu.VMEM((CHUNK,), jnp.float32),
        pltpu.VMEM((ROWS_PER_TILE,), jnp.float32),
        pltpu.VMEM_SHARED((NP,), jnp.float32),
        pltpu.SemaphoreType.DMA,
    ],
)
def _sc_degree(dst_hbm, ones_hbm, z640_hbm, out_hbm,
               dst_v, ones_v, stage_v, deg_sh, sem):
    c = lax.axis_index("c")
    s = lax.axis_index("s")
    wid = c * 16 + s
    row0 = s * ROWS_PER_TILE
    pltpu.sync_copy(dst_hbm.at[wid], dst_v)
    pltpu.sync_copy(ones_hbm, ones_v)
    pltpu.sync_copy(z640_hbm, stage_v)
    pltpu.sync_copy(stage_v, deg_sh.at[pl.ds(row0, ROWS_PER_TILE)])
    plsc.subcore_barrier()

    def body(j, carry):
        pltpu.sync_copy(ones_v, deg_sh.at[dst_v.at[j]], add=True)
        return carry

    lax.fori_loop(0, CPT_DEG, body, 0)
    plsc.subcore_barrier()
    pltpu.sync_copy(deg_sh.at[pl.ds(row0, ROWS_PER_TILE)], stage_v)
    pltpu.sync_copy(stage_v, out_hbm.at[c, pl.ds(row0, ROWS_PER_TILE)])


@functools.partial(
    pl.kernel,
    mesh=_mesh,
    out_type=jax.ShapeDtypeStruct((2, NP, H), jnp.float32),
    scratch_types=[
        pltpu.VMEM((2, CHUNK), jnp.int32),
        pltpu.VMEM((2, CHUNK), jnp.int32),
        pltpu.VMEM((CHUNK, H), jnp.float32),
        pltpu.VMEM((CHUNK, H), jnp.float32),
        pltpu.SemaphoreType.DMA,
        pltpu.SemaphoreType.DMA,
        pltpu.SemaphoreType.DMA,
        pltpu.SemaphoreType.DMA,
        pltpu.SemaphoreType.DMA,
        pltpu.SemaphoreType.DMA,
        pltpu.VMEM_SHARED((NP, H), jnp.float32),
    ],
)
def _sc_aggregate(hs_hbm, src_hbm, dst_hbm, z_hbm, out_hbm,
                  src2, dst2, rows0, rows1, sg0, sg1, si0, si1, sd0, sd1,
                  acc_sh):
    c = lax.axis_index("c")
    s = lax.axis_index("s")
    row0 = s * ROWS_PER_TILE
    NZ = ROWS_PER_TILE // CHUNK

    qh = QPT // 2
    base = (c * 16 + s) * QPT

    pltpu.sync_copy(z_hbm, rows0)
    for k in range(NZ):
        pltpu.sync_copy(rows0, acc_sh.at[pl.ds(row0 + k * CHUNK, CHUNK), :])
    plsc.subcore_barrier()

    def body(g, carry):
        j0 = base + 2 * g
        last = g == qh - 1
        pltpu.make_async_copy(src_hbm.at[base + 1], src2.at[1], si1).wait()
        pltpu.async_copy(hs_hbm.at[src2.at[1]], rows1, sg1)
        pltpu.make_async_copy(hs_hbm.at[src2.at[0]], rows0, sg0).wait()

        @pl.when(jnp.logical_not(last))
        def _():
            pltpu.async_copy(src_hbm.at[j0 + 2], src2.at[0], si0)

        pltpu.make_async_copy(dst_hbm.at[base], dst2.at[0], sd0).wait()
        pltpu.sync_copy(rows0, acc_sh.at[dst2.at[0]], add=True)

        @pl.when(jnp.logical_not(last))
        def _():
            pltpu.async_copy(dst_hbm.at[j0 + 2], dst2.at[0], sd0)
            pltpu.make_async_copy(src_hbm.at[j0 + 2], src2.at[0], si0).wait()
            pltpu.async_copy(hs_hbm.at[src2.at[0]], rows0, sg0)

        pltpu.make_async_copy(hs_hbm.at[src2.at[1]], rows1, sg1).wait()

        @pl.when(jnp.logical_not(last))
        def _():
            pltpu.async_copy(src_hbm.at[j0 + 3], src2.at[1], si1)

        pltpu.make_async_copy(dst_hbm.at[base + 1], dst2.at[1], sd1).wait()
        pltpu.sync_copy(rows1, acc_sh.at[dst2.at[1]], add=True)

        @pl.when(jnp.logical_not(last))
        def _():
            pltpu.async_copy(dst_hbm.at[j0 + 3], dst2.at[1], sd1)

        return carry

    pltpu.sync_copy(src_hbm.at[base], src2.at[0])
    pltpu.async_copy(hs_hbm.at[src2.at[0]], rows0, sg0)
    pltpu.async_copy(dst_hbm.at[base], dst2.at[0], sd0)
    pltpu.async_copy(src_hbm.at[base + 1], src2.at[1], si1)
    pltpu.async_copy(dst_hbm.at[base + 1], dst2.at[1], sd1)
    lax.fori_loop(0, qh, body, 0)

    plsc.subcore_barrier()

    bufs = (rows0, rows1)
    sems = (sg0, sg1)
    for k in range(NZ):
        sl = pl.ds(row0 + k * CHUNK, CHUNK)
        if k >= 2:
            prev = pl.ds(row0 + (k - 2) * CHUNK, CHUNK)
            pltpu.make_async_copy(
                bufs[k % 2], out_hbm.at[c, prev, :], sems[k % 2]).wait()
        pltpu.sync_copy(acc_sh.at[sl, :], bufs[k % 2])
        pltpu.async_copy(bufs[k % 2], out_hbm.at[c, sl, :], sems[k % 2])
    for k in (NZ - 2, NZ - 1):
        sl = pl.ds(row0 + k * CHUNK, CHUNK)
        pltpu.make_async_copy(bufs[k % 2], out_hbm.at[c, sl, :], sems[k % 2]).wait()


def _t1_body(x_ref, w_ref, deg_ref, hs_ref):
    dinv = lax.rsqrt(deg_ref[0, :] + deg_ref[1, :] + 1.0)
    h = jnp.dot(x_ref[...], w_ref[...], preferred_element_type=jnp.float32)
    hs_ref[...] = h * dinv[:, None]


_t1 = pl.pallas_call(
    _t1_body,
    grid=(GRID,),
    in_specs=[
        pl.BlockSpec((RB, D), lambda i: (i, 0)),
        pl.BlockSpec((D, H), lambda i: (0, 0)),
        pl.BlockSpec((2, RB), lambda i: (0, i)),
    ],
    out_specs=pl.BlockSpec((RB, H), lambda i: (i, 0)),
    out_shape=jax.ShapeDtypeStruct((NP, H), jnp.float32),
)


def _t2_body(acc_ref, hs_ref, deg_ref, w_ref, b_ref, out_ref):
    dinv = lax.rsqrt(deg_ref[0, :] + deg_ref[1, :] + 1.0)
    tot = (acc_ref[0] + acc_ref[1] + hs_ref[...]) * dinv[:, None]
    h = jnp.maximum(tot + b_ref[...][None, :], 0.0)
    out_ref[...] = (
        jnp.dot(h, w_ref[...], preferred_element_type=jnp.float32) * dinv[:, None]
    )


_t2 = pl.pallas_call(
    _t2_body,
    grid=(GRID,),
    in_specs=[
        pl.BlockSpec((2, RB, H), lambda i: (0, i, 0)),
        pl.BlockSpec((RB, H), lambda i: (i, 0)),
        pl.BlockSpec((2, RB), lambda i: (0, i)),
        pl.BlockSpec((H, H), lambda i: (0, 0)),
        pl.BlockSpec((H,), lambda i: (0,)),
    ],
    out_specs=pl.BlockSpec((RB, H), lambda i: (i, 0)),
    out_shape=jax.ShapeDtypeStruct((NP, H), jnp.float32),
)


def _t3_body(acc_ref, hs_ref, deg_ref, b_ref, batch_ref, wc_ref, bc_ref,
             out_ref, hg_acc, cnt_acc):
    i = pl.program_id(0)

    @pl.when(i == 0)
    def _():
        hg_acc[...] = jnp.zeros_like(hg_acc)
        cnt_acc[...] = jnp.zeros_like(cnt_acc)

    dinv = lax.rsqrt(deg_ref[0, :] + deg_ref[1, :] + 1.0)
    tot = (acc_ref[0] + acc_ref[1] + hs_ref[...]) * dinv[:, None]
    h = jnp.maximum(tot + b_ref[...][None, :], 0.0)
    onehot = (
        batch_ref[0, 0, :][None, :]
        == lax.broadcasted_iota(jnp.int32, (G, RB), 0)
    ).astype(jnp.float32)
    hg_acc[...] += jnp.dot(onehot, h, preferred_element_type=jnp.float32)
    cnt = jnp.sum(onehot, axis=1, keepdims=True)
    cnt_acc[...] += jnp.broadcast_to(cnt, (G, H))

    @pl.when(i == GRID - 1)
    def _():
        hg = hg_acc[...] / jnp.maximum(cnt_acc[...], 1.0)
        out_ref[...] = (
            jnp.dot(hg, wc_ref[...], preferred_element_type=jnp.float32)
            + bc_ref[...][None, :]
        )


_t3 = pl.pallas_call(
    _t3_body,
    grid=(GRID,),
    in_specs=[
        pl.BlockSpec((2, RB, H), lambda i: (0, i, 0)),
        pl.BlockSpec((RB, H), lambda i: (i, 0)),
        pl.BlockSpec((2, RB), lambda i: (0, i)),
        pl.BlockSpec((H,), lambda i: (0,)),
        pl.BlockSpec((1, 1, RB), lambda i: (i, 0, 0)),
        pl.BlockSpec((H, H), lambda i: (0, 0)),
        pl.BlockSpec((H,), lambda i: (0,)),
    ],
    out_specs=pl.BlockSpec((G, H), lambda i: (0, 0)),
    out_shape=jax.ShapeDtypeStruct((G, H), jnp.float32),
    scratch_shapes=[
        pltpu.VMEM((G, H), jnp.float32),
        pltpu.VMEM((G, H), jnp.float32),
    ],
)


def kernel(x, edge_index, batch, W1, b1, W2, b2, Wc, bc):
    src = edge_index[0]
    dst = edge_index[1]
    pad_e = EP - E
    pad_dst = N + (jnp.arange(pad_e, dtype=jnp.int32) % (NP - N))
    pad_src = N + ((jnp.arange(pad_e, dtype=jnp.int32) + 97) % (NP - N))
    src_p = jnp.concatenate([src, pad_src]).reshape(NCHUNKS, CHUNK)
    dst_p = jnp.concatenate([dst, pad_dst]).reshape(NCHUNKS, CHUNK)
    dst_p32 = dst_p.reshape(32, CPT_DEG, CHUNK)
    x_p = jnp.pad(x, ((0, NP - N), (0, 0)))
    batch_p = jnp.pad(batch, (0, NP - N), constant_values=G).reshape(GRID, 1, RB)
    wc_p = jnp.pad(Wc, ((0, 0), (0, H - C)))
    bc_p = jnp.pad(bc, (0, H - C))
    ones128 = jnp.ones((CHUNK,), jnp.float32)
    z640 = jnp.zeros((ROWS_PER_TILE,), jnp.float32)
    z128 = jnp.zeros((CHUNK, H), jnp.float32)

    deg2 = _sc_degree(dst_p32, ones128, z640)
    hs1 = _t1(x_p, W1, deg2)
    acc1 = _sc_aggregate(hs1, src_p, dst_p, z128)
    hs2 = _t2(acc1, hs1, deg2, W2, b1)
    acc2 = _sc_aggregate(hs2, src_p, dst_p, z128)
    out = _t3(acc2, hs2, deg2, b2, batch_p, wc_p, bc_p)
    return out[:, :C]


# speedup vs baseline: 3.7988x; 1.0144x over previous
import functools

import jax
import jax.numpy as jnp
from jax import lax
from jax.experimental import pallas as pl
from jax.experimental.pallas import tpu as pltpu
from jax.experimental.pallas import tpu_sc as plsc

N = 10000
E = 320000
D = 128
H = 128
C = 16
G = 16

NP = 10240
ROWS_PER_TILE = 640
CHUNK = 128
NCHUNKS = 2560
EP = NCHUNKS * CHUNK
QPT = 80
CPT_DEG = 80
RB = 1024
GRID = NP // RB

_mesh = plsc.VectorSubcoreMesh(core_axis_name="c", subcore_axis_name="s")


@functools.partial(
    pl.kernel,
    mesh=_mesh,
    out_type=jax.ShapeDtypeStruct((2, NP), jnp.float32),
    scratch_types=[
        pltpu.VMEM((CPT_DEG, CHUNK), jnp.int32),
        pltpu.VMEM((CHUNK,), jnp.float32),
        pltpu.VMEM((ROWS_PER_TILE,), jnp.float32),
        pltpu.VMEM_SHARED((NP,), jnp.float32),
        pltpu.SemaphoreType.DMA,
    ],
)
def _sc_degree(dst_hbm, ones_hbm, z640_hbm, out_hbm,
               dst_v, ones_v, stage_v, deg_sh, sem):
    c = lax.axis_index("c")
    s = lax.axis_index("s")
    wid = c * 16 + s
    row0 = s * ROWS_PER_TILE
    pltpu.sync_copy(dst_hbm.at[wid], dst_v)
    pltpu.sync_copy(ones_hbm, ones_v)
    pltpu.sync_copy(z640_hbm, stage_v)
    pltpu.sync_copy(stage_v, deg_sh.at[pl.ds(row0, ROWS_PER_TILE)])
    plsc.subcore_barrier()

    def body(b, carry):
        for u in range(8):
            pltpu.async_copy(ones_v, deg_sh.at[dst_v.at[b * 8 + u]], sem, add=True)
        for u in range(8):
            pltpu.make_async_copy(ones_v, deg_sh.at[dst_v.at[b * 8 + u]], sem).wait()
        return carry

    lax.fori_loop(0, CPT_DEG // 8, body, 0)
    plsc.subcore_barrier()
    pltpu.sync_copy(deg_sh.at[pl.ds(row0, ROWS_PER_TILE)], stage_v)
    pltpu.sync_copy(stage_v, out_hbm.at[c, pl.ds(row0, ROWS_PER_TILE)])


@functools.partial(
    pl.kernel,
    mesh=_mesh,
    out_type=jax.ShapeDtypeStruct((2, NP, H), jnp.float32),
    scratch_types=[
        pltpu.VMEM((2, CHUNK), jnp.int32),
        pltpu.VMEM((2, CHUNK), jnp.int32),
        pltpu.VMEM((CHUNK, H), jnp.float32),
        pltpu.VMEM((CHUNK, H), jnp.float32),
        pltpu.SemaphoreType.DMA,
        pltpu.SemaphoreType.DMA,
        pltpu.SemaphoreType.DMA,
        pltpu.SemaphoreType.DMA,
        pltpu.SemaphoreType.DMA,
        pltpu.SemaphoreType.DMA,
        pltpu.SemaphoreType.DMA,
        pltpu.SemaphoreType.DMA,
        pltpu.VMEM_SHARED((NP, H), jnp.float32),
    ],
)
def _sc_aggregate(hs_hbm, src_hbm, dst_hbm, z_hbm, out_hbm,
                  src2, dst2, rows0, rows1, sg0, sg1, si0, si1, sd0, sd1,
                  ss0, ss1, acc_sh):
    c = lax.axis_index("c")
    s = lax.axis_index("s")
    row0 = s * ROWS_PER_TILE
    NZ = ROWS_PER_TILE // CHUNK

    qh = QPT // 2
    base = (c * 16 + s) * QPT

    pltpu.sync_copy(z_hbm, rows0)
    for k in range(NZ):
        pltpu.sync_copy(rows0, acc_sh.at[pl.ds(row0 + k * CHUNK, CHUNK), :])
    plsc.subcore_barrier()

    def body(g, carry):
        j0 = base + 2 * g
        notlast = g < qh - 1

        pltpu.make_async_copy(src_hbm.at[base + 1], src2.at[1], si1).wait()

        @pl.when(g > 0)
        def _():
            pltpu.make_async_copy(rows1, acc_sh.at[dst2.at[1]], ss1).wait()
            pltpu.async_copy(dst_hbm.at[j0 + 1], dst2.at[1], sd1)

        pltpu.async_copy(hs_hbm.at[src2.at[1]], rows1, sg1)
        pltpu.make_async_copy(hs_hbm.at[src2.at[0]], rows0, sg0).wait()

        @pl.when(notlast)
        def _():
            pltpu.async_copy(src_hbm.at[j0 + 2], src2.at[0], si0)

        pltpu.make_async_copy(dst_hbm.at[base], dst2.at[0], sd0).wait()
        pltpu.async_copy(rows0, acc_sh.at[dst2.at[0]], ss0, add=True)

        @pl.when(notlast)
        def _():
            pltpu.make_async_copy(rows0, acc_sh.at[dst2.at[0]], ss0).wait()
            pltpu.async_copy(dst_hbm.at[j0 + 2], dst2.at[0], sd0)
            pltpu.make_async_copy(src_hbm.at[j0 + 2], src2.at[0], si0).wait()
            pltpu.async_copy(hs_hbm.at[src2.at[0]], rows0, sg0)

        pltpu.make_async_copy(hs_hbm.at[src2.at[1]], rows1, sg1).wait()
        pltpu.make_async_copy(dst_hbm.at[base + 1], dst2.at[1], sd1).wait()
        pltpu.async_copy(rows1, acc_sh.at[dst2.at[1]], ss1, add=True)

        @pl.when(notlast)
        def _():
            pltpu.async_copy(src_hbm.at[j0 + 3], src2.at[1], si1)

        return carry

    pltpu.sync_copy(src_hbm.at[base], src2.at[0])
    pltpu.async_copy(hs_hbm.at[src2.at[0]], rows0, sg0)
    pltpu.async_copy(dst_hbm.at[base], dst2.at[0], sd0)
    pltpu.async_copy(src_hbm.at[base + 1], src2.at[1], si1)
    pltpu.async_copy(dst_hbm.at[base + 1], dst2.at[1], sd1)
    lax.fori_loop(0, qh, body, 0)

    pltpu.make_async_copy(rows0, acc_sh.at[dst2.at[0]], ss0).wait()
    pltpu.make_async_copy(rows1, acc_sh.at[dst2.at[1]], ss1).wait()
    plsc.subcore_barrier()

    bufs = (rows0, rows1)
    sems = (sg0, sg1)
    for k in range(NZ):
        sl = pl.ds(row0 + k * CHUNK, CHUNK)
        if k >= 2:
            prev = pl.ds(row0 + (k - 2) * CHUNK, CHUNK)
            pltpu.make_async_copy(
                bufs[k % 2], out_hbm.at[c, prev, :], sems[k % 2]).wait()
        pltpu.sync_copy(acc_sh.at[sl, :], bufs[k % 2])
        pltpu.async_copy(bufs[k % 2], out_hbm.at[c, sl, :], sems[k % 2])
    for k in (NZ - 2, NZ - 1):
        sl = pl.ds(row0 + k * CHUNK, CHUNK)
        pltpu.make_async_copy(bufs[k % 2], out_hbm.at[c, sl, :], sems[k % 2]).wait()


def _t1_body(x_ref, w_ref, deg_ref, hs_ref):
    dinv = lax.rsqrt(deg_ref[0, :] + deg_ref[1, :] + 1.0)
    h = jnp.dot(x_ref[...], w_ref[...], preferred_element_type=jnp.float32)
    hs_ref[...] = h * dinv[:, None]


_t1 = pl.pallas_call(
    _t1_body,
    grid=(GRID,),
    in_specs=[
        pl.BlockSpec((RB, D), lambda i: (i, 0)),
        pl.BlockSpec((D, H), lambda i: (0, 0)),
        pl.BlockSpec((2, RB), lambda i: (0, i)),
    ],
    out_specs=pl.BlockSpec((RB, H), lambda i: (i, 0)),
    out_shape=jax.ShapeDtypeStruct((NP, H), jnp.float32),
)


def _t2_body(acc_ref, hs_ref, deg_ref, w_ref, b_ref, out_ref):
    dinv = lax.rsqrt(deg_ref[0, :] + deg_ref[1, :] + 1.0)
    tot = (acc_ref[0] + acc_ref[1] + hs_ref[...]) * dinv[:, None]
    h = jnp.maximum(tot + b_ref[...][None, :], 0.0)
    out_ref[...] = (
        jnp.dot(h, w_ref[...], preferred_element_type=jnp.float32) * dinv[:, None]
    )


_t2 = pl.pallas_call(
    _t2_body,
    grid=(GRID,),
    in_specs=[
        pl.BlockSpec((2, RB, H), lambda i: (0, i, 0)),
        pl.BlockSpec((RB, H), lambda i: (i, 0)),
        pl.BlockSpec((2, RB), lambda i: (0, i)),
        pl.BlockSpec((H, H), lambda i: (0, 0)),
        pl.BlockSpec((H,), lambda i: (0,)),
    ],
    out_specs=pl.BlockSpec((RB, H), lambda i: (i, 0)),
    out_shape=jax.ShapeDtypeStruct((NP, H), jnp.float32),
)


def _t3_body(acc_ref, hs_ref, deg_ref, b_ref, batch_ref, wc_ref, bc_ref,
             out_ref, hg_acc, cnt_acc):
    i = pl.program_id(0)

    @pl.when(i == 0)
    def _():
        hg_acc[...] = jnp.zeros_like(hg_acc)
        cnt_acc[...] = jnp.zeros_like(cnt_acc)

    dinv = lax.rsqrt(deg_ref[0, :] + deg_ref[1, :] + 1.0)
    tot = (acc_ref[0] + acc_ref[1] + hs_ref[...]) * dinv[:, None]
    h = jnp.maximum(tot + b_ref[...][None, :], 0.0)
    onehot = (
        batch_ref[0, 0, :][None, :]
        == lax.broadcasted_iota(jnp.int32, (G, RB), 0)
    ).astype(jnp.float32)
    hg_acc[...] += jnp.dot(onehot, h, preferred_element_type=jnp.float32)
    cnt = jnp.sum(onehot, axis=1, keepdims=True)
    cnt_acc[...] += jnp.broadcast_to(cnt, (G, H))

    @pl.when(i == GRID - 1)
    def _():
        hg = hg_acc[...] / jnp.maximum(cnt_acc[...], 1.0)
        out_ref[...] = (
            jnp.dot(hg, wc_ref[...], preferred_element_type=jnp.float32)
            + bc_ref[...][None, :]
        )


_t3 = pl.pallas_call(
    _t3_body,
    grid=(GRID,),
    in_specs=[
        pl.BlockSpec((2, RB, H), lambda i: (0, i, 0)),
        pl.BlockSpec((RB, H), lambda i: (i, 0)),
        pl.BlockSpec((2, RB), lambda i: (0, i)),
        pl.BlockSpec((H,), lambda i: (0,)),
        pl.BlockSpec((1, 1, RB), lambda i: (i, 0, 0)),
        pl.BlockSpec((H, H), lambda i: (0, 0)),
        pl.BlockSpec((H,), lambda i: (0,)),
    ],
    out_specs=pl.BlockSpec((G, H), lambda i: (0, 0)),
    out_shape=jax.ShapeDtypeStruct((G, H), jnp.float32),
    scratch_shapes=[
        pltpu.VMEM((G, H), jnp.float32),
        pltpu.VMEM((G, H), jnp.float32),
    ],
)


def kernel(x, edge_index, batch, W1, b1, W2, b2, Wc, bc):
    src = edge_index[0]
    dst = edge_index[1]
    pad_e = EP - E
    pad_dst = N + (jnp.arange(pad_e, dtype=jnp.int32) % (NP - N))
    pad_src = N + ((jnp.arange(pad_e, dtype=jnp.int32) + 97) % (NP - N))
    src_p = jnp.concatenate([src, pad_src]).reshape(NCHUNKS, CHUNK)
    dst_p = jnp.concatenate([dst, pad_dst]).reshape(NCHUNKS, CHUNK)
    dst_p32 = dst_p.reshape(32, CPT_DEG, CHUNK)
    x_p = jnp.pad(x, ((0, NP - N), (0, 0)))
    batch_p = jnp.pad(batch, (0, NP - N), constant_values=G).reshape(GRID, 1, RB)
    wc_p = jnp.pad(Wc, ((0, 0), (0, H - C)))
    bc_p = jnp.pad(bc, (0, H - C))
    ones128 = jnp.ones((CHUNK,), jnp.float32)
    z640 = jnp.zeros((ROWS_PER_TILE,), jnp.float32)
    z128 = jnp.zeros((CHUNK, H), jnp.float32)

    deg2 = _sc_degree(dst_p32, ones128, z640)
    hs1 = _t1(x_p, W1, deg2)
    acc1 = _sc_aggregate(hs1, src_p, dst_p, z128)
    hs2 = _t2(acc1, hs1, deg2, W2, b1)
    acc2 = _sc_aggregate(hs2, src_p, dst_p, z128)
    out = _t3(acc2, hs2, deg2, b2, batch_p, wc_p, bc_p)
    return out[:, :C]
